# baseline (speedup 1.0000x reference)
import functools

import jax
import jax.numpy as jnp
from jax import lax
from jax.experimental import pallas as pl
from jax.experimental.pallas import tpu as pltpu
from jax.experimental.pallas import tpu_sc as plsc

_THRESHOLD = 0.9
_L = 16
_BIG = 0x7FFFFFFF


def _sim_body(slots_ref, simp_ref, rmax_ref, rarg_ref):
    n = slots_ref.shape[1]
    npad = simp_ref.shape[2]
    x = slots_ref[0]
    nrm = jnp.sqrt(jnp.sum(x * x, axis=1, keepdims=True))
    xn = x / jnp.maximum(nrm, 1e-12)
    sim = lax.dot_general(
        xn, xn, (((1,), (1,)), ((), ())),
        preferred_element_type=jnp.float32,
    )
    row_i = lax.broadcasted_iota(jnp.int32, (n, n), 0)
    col_i = lax.broadcasted_iota(jnp.int32, (n, n), 1)
    sim = jnp.where(row_i == col_i, sim - 2.0, sim)

    cmax = jnp.max(sim, axis=0, keepdims=True)
    pad = npad - n
    rmax_ref[0] = jnp.concatenate(
        [cmax, jnp.full((1, pad), -2.0, jnp.float32)], axis=1)

    @pl.when(jnp.max(cmax) > _THRESHOLD)
    def _():
        carg = jnp.min(jnp.where(sim == cmax, row_i, jnp.int32(_BIG)),
                       axis=0, keepdims=True)
        simp_ref[0] = jnp.concatenate(
            [sim, jnp.full((n, pad), -2.0, jnp.float32)], axis=1)
        rarg_ref[0] = jnp.concatenate(
            [carg, jnp.zeros((1, pad), jnp.int32)], axis=1)


def _greedy_body(simp, rmax0, rarg0, slots, mt_out, merged,
                 rmax_v, rarg_v, alive_v, mt_v, row_v, cnt_v, acc_v, srow_v,
                 mflag, sem):
    b_total, n_rows, npad = simp.shape
    d = slots.shape[2]
    nchunks = npad // _L
    dchunks = d // _L
    wid = lax.axis_index("s") * 2 + lax.axis_index("c")
    lane = lax.iota(jnp.int32, _L)
    lane0 = lane == 0

    def splat_i(s):
        return jnp.full((_L,), s, jnp.int32)

    def splat_f(s):
        return jnp.full((_L,), s, jnp.float32)

    def gather_i(ref, idx):
        return jnp.min(plsc.load_gather(ref, [splat_i(idx)]))

    @pl.when(wid < b_total)
    def _():
        b = wid
        pltpu.async_copy(rmax0.at[b], rmax_v, sem).wait()
        pltpu.async_copy(rarg0.at[b], rarg_v, sem).wait()

        def init_chunk(j, _):
            idx = lane + j * _L
            alive_v[pl.ds(j * _L, _L)] = jnp.where(
                idx < n_rows, 1.0, 0.0).astype(jnp.float32)
            mt_v[pl.ds(j * _L, _L)] = idx
            return 0

        lax.fori_loop(0, nchunks, init_chunk, 0)

        def global_argmax():
            def step(j, carry):
                vm, vi = carry
                v = rmax_v[pl.ds(j * _L, _L)]
                idx = lane + j * _L
                upd = v > vm
                return jnp.where(upd, v, vm), jnp.where(upd, idx, vi)

            vm, vi = lax.fori_loop(
                0, nchunks, step,
                (splat_f(-3.0), jnp.zeros((_L,), jnp.int32)))
            m = jnp.max(vm)
            r = jnp.min(jnp.where(vm == m, vi, jnp.int32(_BIG)))
            return m, r

        def masked_row_argmax():
            def step(k, carry):
                vm, vi = carry
                v = row_v[pl.ds(k * _L, _L)]
                a = alive_v[pl.ds(k * _L, _L)]
                w = jnp.where(a > 0.5, v, -2.0)
                idx = lane + k * _L
                upd = w > vm
                return jnp.where(upd, w, vm), jnp.where(upd, idx, vi)

            vm, vi = lax.fori_loop(
                0, nchunks, step,
                (splat_f(-3.0), jnp.zeros((_L,), jnp.int32)))
            m = jnp.max(vm)
            c = jnp.min(jnp.where(vm == m, vi, jnp.int32(_BIG)))
            return m, c

        def cond(st):
            return st != 0

        def body(st):
            m, r = global_argmax()
            go = m > _THRESHOLD
            _do_merge(go, r)
            return go.astype(jnp.int32)

        def _do_merge(go, r):
            @pl.when(go)
            def _():
                mflag[0] = jnp.int32(1)
                c = gather_i(rarg_v, r)
                src = jnp.maximum(r, c)
                tgt = jnp.minimum(r, c)
                plsc.store_scatter(mt_v, [splat_i(src)], splat_i(tgt),
                                   mask=lane0)
                plsc.store_scatter(alive_v, [splat_i(src)], splat_f(0.0),
                                   mask=lane0)
                plsc.store_scatter(rmax_v, [splat_i(src)], splat_f(-2.0),
                                   mask=lane0)

                def scan_chunk(j, _):
                    rarg_c = rarg_v[pl.ds(j * _L, _L)]
                    alive_c = alive_v[pl.ds(j * _L, _L)]
                    need = (rarg_c == src) & (alive_c > 0.5)
                    cnt = plsc.all_reduce_population_count(need)

                    @pl.when(jnp.max(cnt) > 0)
                    def _():
                        def lane_step(l, _):
                            i = j * _L + l
                            hit = (gather_i(rarg_v, i) == src) & (
                                jnp.min(plsc.load_gather(
                                    alive_v, [splat_i(i)])) > 0.5)

                            @pl.when(hit)
                            def _():
                                pltpu.async_copy(simp.at[b, i], row_v,
                                                 sem).wait()
                                nm, nc = masked_row_argmax()
                                plsc.store_scatter(
                                    rmax_v, [splat_i(i)], splat_f(nm),
                                    mask=lane0)
                                plsc.store_scatter(
                                    rarg_v, [splat_i(i)], splat_i(nc),
                                    mask=lane0)

                            return 0

                        lax.fori_loop(0, _L, lane_step, 0)

                    return 0

                lax.fori_loop(0, nchunks, scan_chunk, 0)

        mflag[0] = jnp.int32(0)
        lax.while_loop(cond, body, jnp.int32(1))
        pltpu.async_copy(mt_v, mt_out.at[b], sem).wait()

        @pl.when(mflag[0] == 0)
        def _():
            pltpu.async_copy(slots.at[b], merged.at[b], sem).wait()

        @pl.when(mflag[0] != 0)
        def _():
            ones = jnp.ones((_L,), jnp.float32)
            zeros = jnp.zeros((_L,), jnp.float32)

            def zc(j, _):
                cnt_v[pl.ds(j * _L, _L)] = zeros
                return 0

            lax.fori_loop(0, nchunks, zc, 0)

            def cc(j, _):
                mtc = mt_v[pl.ds(j * _L, _L)]
                valid = (lane + j * _L) < n_rows
                plsc.addupdate_scatter(cnt_v, [mtc], ones, mask=valid)
                return 0

            lax.fori_loop(0, nchunks, cc, 0)

            rpp = acc_v.shape[0]
            for p in range((n_rows + rpp - 1) // rpp):
                base = p * rpp
                rows_out = min(rpp, n_rows - base)

                def za(r, _):
                    def zk(k, _):
                        acc_v[r, pl.ds(k * _L, _L)] = zeros
                        return 0

                    lax.fori_loop(0, dchunks, zk, 0)
                    return 0

                lax.fori_loop(0, rpp, za, 0)

                def si(i, _):
                    t = gather_i(mt_v, i)
                    hit = (t >= base) & (t < base + rpp)

                    @pl.when(hit)
                    def _():
                        pltpu.async_copy(slots.at[b, i], srow_v, sem).wait()
                        rel = t - base

                        def ak(k, _):
                            sl = pl.ds(k * _L, _L)
                            acc_v[rel, sl] = acc_v[rel, sl] + srow_v[sl]
                            return 0

                        lax.fori_loop(0, dchunks, ak, 0)

                    return 0

                lax.fori_loop(0, n_rows, si, 0)

                def dv(r, _):
                    cv = plsc.load_gather(cnt_v, [splat_i(base + r)])
                    scale = ones / jnp.maximum(cv, 1.0)

                    def mk(k, _):
                        sl = pl.ds(k * _L, _L)
                        acc_v[r, sl] = acc_v[r, sl] * scale
                        return 0

                    lax.fori_loop(0, dchunks, mk, 0)
                    return 0

                lax.fori_loop(0, rows_out, dv, 0)
                pltpu.async_copy(acc_v.at[pl.ds(0, rows_out)],
                                 merged.at[b, pl.ds(base, rows_out)],
                                 sem).wait()


def kernel(slots):
    b, n, d = slots.shape
    npad = ((n + _L - 1) // _L) * _L

    simp, rmax, rarg = pl.pallas_call(
        _sim_body,
        grid=(b,),
        in_specs=[pl.BlockSpec((1, n, d), lambda i: (i, 0, 0))],
        out_specs=[
            pl.BlockSpec((1, n, npad), lambda i: (i, 0, 0)),
            pl.BlockSpec((1, 1, npad), lambda i: (i, 0, 0)),
            pl.BlockSpec((1, 1, npad), lambda i: (i, 0, 0)),
        ],
        out_shape=[
            jax.ShapeDtypeStruct((b, n, npad), jnp.float32),
            jax.ShapeDtypeStruct((b, 1, npad), jnp.float32),
            jax.ShapeDtypeStruct((b, 1, npad), jnp.int32),
        ],
    )(slots)

    mesh = plsc.VectorSubcoreMesh(core_axis_name="c", subcore_axis_name="s",
                                  num_cores=2, num_subcores=16)
    acc_rows = 256
    mt_pad, merged = pl.kernel(
        _greedy_body,
        out_type=[
            jax.ShapeDtypeStruct((b, npad), jnp.int32),
            jax.ShapeDtypeStruct((b, n, d), jnp.float32),
        ],
        mesh=mesh,
        compiler_params=pltpu.CompilerParams(needs_layout_passes=False),
        scratch_types=[
            pltpu.VMEM((npad,), jnp.float32),
            pltpu.VMEM((npad,), jnp.int32),
            pltpu.VMEM((npad,), jnp.float32),
            pltpu.VMEM((npad,), jnp.int32),
            pltpu.VMEM((npad,), jnp.float32),
            pltpu.VMEM((npad,), jnp.float32),
            pltpu.VMEM((acc_rows, d), jnp.float32),
            pltpu.VMEM((d,), jnp.float32),
            pltpu.SMEM((1,), jnp.int32),
            pltpu.SemaphoreType.DMA,
        ],
    )(simp, rmax.reshape(b, npad), rarg.reshape(b, npad), slots)

    return merged, mt_pad[:, :n]


# speedup vs baseline: 2.8869x; 2.8869x over previous
import functools

import jax
import jax.numpy as jnp
from jax import lax
from jax.experimental import pallas as pl
from jax.experimental.pallas import tpu as pltpu
from jax.experimental.pallas import tpu_sc as plsc

_THRESHOLD = 0.9
_L = 16
_BIG = 0x7FFFFFFF


def _sim_body(slots_ref, simp_ref, rmax_ref, rarg_ref):
    n = slots_ref.shape[1]
    npad = simp_ref.shape[2]
    x = slots_ref[0]
    nrm = jnp.sqrt(jnp.sum(x * x, axis=1, keepdims=True))
    xn = x / jnp.maximum(nrm, 1e-12)
    sim = lax.dot_general(
        xn, xn, (((1,), (1,)), ((), ())),
        preferred_element_type=jnp.float32,
    )
    row_i = lax.broadcasted_iota(jnp.int32, (n, n), 0)
    col_i = lax.broadcasted_iota(jnp.int32, (n, n), 1)
    sim = jnp.where(row_i == col_i, sim - 2.0, sim)

    cmax = jnp.max(sim, axis=0, keepdims=True)
    pad = npad - n
    rmax_ref[0] = jnp.concatenate(
        [cmax, jnp.full((1, pad), -2.0, jnp.float32)], axis=1)

    @pl.when(jnp.max(cmax) > _THRESHOLD)
    def _():
        carg = jnp.min(jnp.where(sim == cmax, row_i, jnp.int32(_BIG)),
                       axis=0, keepdims=True)
        simp_ref[0] = jnp.concatenate(
            [sim, jnp.full((n, pad), -2.0, jnp.float32)], axis=1)
        rarg_ref[0] = jnp.concatenate(
            [carg, jnp.zeros((1, pad), jnp.int32)], axis=1)


def _greedy_body(simp, rmax0, rarg0, slots, mt_out, merged,
                 rmax_v, rarg_v, alive_v, mt_v, row_v, cnt_v, acc_v, srow_v,
                 mflag, sem):
    b_total, n_rows, npad = simp.shape
    d = slots.shape[2]
    nchunks = npad // _L
    dchunks = d // _L
    wid = lax.axis_index("s") * 2 + lax.axis_index("c")
    lane = lax.iota(jnp.int32, _L)
    lane0 = lane == 0

    def splat_i(s):
        return jnp.full((_L,), s, jnp.int32)

    def splat_f(s):
        return jnp.full((_L,), s, jnp.float32)

    def gather_i(ref, idx):
        return jnp.min(plsc.load_gather(ref, [splat_i(idx)]))

    @pl.when(wid < b_total)
    def _():
        b = wid
        pltpu.async_copy(rmax0.at[b], rmax_v, sem).wait()
        pltpu.async_copy(rarg0.at[b], rarg_v, sem).wait()

        def init_chunk(j, _):
            idx = lane + j * _L
            alive_v[pl.ds(j * _L, _L)] = jnp.where(
                idx < n_rows, 1.0, 0.0).astype(jnp.float32)
            mt_v[pl.ds(j * _L, _L)] = idx
            return 0

        lax.fori_loop(0, nchunks, init_chunk, 0)

        def global_argmax():
            def step(j, carry):
                vm, vi = carry
                v = rmax_v[pl.ds(j * _L, _L)]
                idx = lane + j * _L
                upd = v > vm
                return jnp.where(upd, v, vm), jnp.where(upd, idx, vi)

            vm, vi = lax.fori_loop(
                0, nchunks, step,
                (splat_f(-3.0), jnp.zeros((_L,), jnp.int32)))
            m = jnp.max(vm)
            r = jnp.min(jnp.where(vm == m, vi, jnp.int32(_BIG)))
            return m, r

        def masked_row_argmax():
            def step(k, carry):
                vm, vi = carry
                v = row_v[pl.ds(k * _L, _L)]
                a = alive_v[pl.ds(k * _L, _L)]
                w = jnp.where(a > 0.5, v, -2.0)
                idx = lane + k * _L
                upd = w > vm
                return jnp.where(upd, w, vm), jnp.where(upd, idx, vi)

            vm, vi = lax.fori_loop(
                0, nchunks, step,
                (splat_f(-3.0), jnp.zeros((_L,), jnp.int32)))
            m = jnp.max(vm)
            c = jnp.min(jnp.where(vm == m, vi, jnp.int32(_BIG)))
            return m, c

        def cond(st):
            return st != 0

        def body(st):
            m, r = global_argmax()
            go = m > _THRESHOLD
            _do_merge(go, r)
            return go.astype(jnp.int32)

        def _do_merge(go, r):
            @pl.when(go)
            def _():
                mflag[0] = jnp.int32(1)
                c = gather_i(rarg_v, r)
                src = jnp.maximum(r, c)
                tgt = jnp.minimum(r, c)
                plsc.store_scatter(mt_v, [splat_i(src)], splat_i(tgt),
                                   mask=lane0)
                plsc.store_scatter(alive_v, [splat_i(src)], splat_f(0.0),
                                   mask=lane0)
                plsc.store_scatter(rmax_v, [splat_i(src)], splat_f(-2.0),
                                   mask=lane0)

                def scan_chunk(j, _):
                    rarg_c = rarg_v[pl.ds(j * _L, _L)]
                    alive_c = alive_v[pl.ds(j * _L, _L)]
                    need = (rarg_c == src) & (alive_c > 0.5)
                    cnt = plsc.all_reduce_population_count(need)

                    @pl.when(jnp.max(cnt) > 0)
                    def _():
                        def lane_step(l, _):
                            i = j * _L + l
                            hit = (gather_i(rarg_v, i) == src) & (
                                jnp.min(plsc.load_gather(
                                    alive_v, [splat_i(i)])) > 0.5)

                            @pl.when(hit)
                            def _():
                                pltpu.async_copy(simp.at[b, i], row_v,
                                                 sem).wait()
                                nm, nc = masked_row_argmax()
                                plsc.store_scatter(
                                    rmax_v, [splat_i(i)], splat_f(nm),
                                    mask=lane0)
                                plsc.store_scatter(
                                    rarg_v, [splat_i(i)], splat_i(nc),
                                    mask=lane0)

                            return 0

                        lax.fori_loop(0, _L, lane_step, 0)

                    return 0

                lax.fori_loop(0, nchunks, scan_chunk, 0)

        mflag[0] = jnp.int32(0)
        lax.while_loop(cond, body, jnp.int32(1))
        pltpu.async_copy(mt_v, mt_out.at[b], sem).wait()

        @pl.when(mflag[0] == 0)
        def _():
            crow = 200
            for cidx in range(n_rows // crow):
                src = slots.at[b, pl.ds(cidx * crow, crow)]
                dst = merged.at[b, pl.ds(cidx * crow, crow)]
                stage = acc_v.at[pl.ds(0, crow)]
                pltpu.async_copy(src, stage, sem).wait()
                pltpu.async_copy(stage, dst, sem).wait()

        @pl.when(mflag[0] != 0)
        def _():
            ones = jnp.ones((_L,), jnp.float32)
            zeros = jnp.zeros((_L,), jnp.float32)

            def zc(j, _):
                cnt_v[pl.ds(j * _L, _L)] = zeros
                return 0

            lax.fori_loop(0, nchunks, zc, 0)

            def cc(j, _):
                mtc = mt_v[pl.ds(j * _L, _L)]
                valid = (lane + j * _L) < n_rows
                plsc.addupdate_scatter(cnt_v, [mtc], ones, mask=valid)
                return 0

            lax.fori_loop(0, nchunks, cc, 0)

            rpp = acc_v.shape[0]
            for p in range((n_rows + rpp - 1) // rpp):
                base = p * rpp
                rows_out = min(rpp, n_rows - base)

                def za(r, _):
                    def zk(k, _):
                        acc_v[r, pl.ds(k * _L, _L)] = zeros
                        return 0

                    lax.fori_loop(0, dchunks, zk, 0)
                    return 0

                lax.fori_loop(0, rpp, za, 0)

                def si(i, _):
                    t = gather_i(mt_v, i)
                    hit = (t >= base) & (t < base + rpp)

                    @pl.when(hit)
                    def _():
                        pltpu.async_copy(slots.at[b, i], srow_v, sem).wait()
                        rel = t - base

                        def ak(k, _):
                            sl = pl.ds(k * _L, _L)
                            acc_v[rel, sl] = acc_v[rel, sl] + srow_v[sl]
                            return 0

                        lax.fori_loop(0, dchunks, ak, 0)

                    return 0

                lax.fori_loop(0, n_rows, si, 0)

                def dv(r, _):
                    cv = plsc.load_gather(cnt_v, [splat_i(base + r)])
                    scale = ones / jnp.maximum(cv, 1.0)

                    def mk(k, _):
                        sl = pl.ds(k * _L, _L)
                        acc_v[r, sl] = acc_v[r, sl] * scale
                        return 0

                    lax.fori_loop(0, dchunks, mk, 0)
                    return 0

                lax.fori_loop(0, rows_out, dv, 0)
                pltpu.async_copy(acc_v.at[pl.ds(0, rows_out)],
                                 merged.at[b, pl.ds(base, rows_out)],
                                 sem).wait()


def kernel(slots):
    b, n, d = slots.shape
    npad = ((n + _L - 1) // _L) * _L

    simp, rmax, rarg = pl.pallas_call(
        _sim_body,
        grid=(b,),
        in_specs=[pl.BlockSpec((1, n, d), lambda i: (i, 0, 0))],
        out_specs=[
            pl.BlockSpec((1, n, npad), lambda i: (i, 0, 0)),
            pl.BlockSpec((1, 1, npad), lambda i: (i, 0, 0)),
            pl.BlockSpec((1, 1, npad), lambda i: (i, 0, 0)),
        ],
        out_shape=[
            jax.ShapeDtypeStruct((b, n, npad), jnp.float32),
            jax.ShapeDtypeStruct((b, 1, npad), jnp.float32),
            jax.ShapeDtypeStruct((b, 1, npad), jnp.int32),
        ],
    )(slots)

    mesh = plsc.VectorSubcoreMesh(core_axis_name="c", subcore_axis_name="s",
                                  num_cores=2, num_subcores=16)
    acc_rows = 256
    mt_pad, merged = pl.kernel(
        _greedy_body,
        out_type=[
            jax.ShapeDtypeStruct((b, npad), jnp.int32),
            jax.ShapeDtypeStruct((b, n, d), jnp.float32),
        ],
        mesh=mesh,
        compiler_params=pltpu.CompilerParams(needs_layout_passes=False),
        scratch_types=[
            pltpu.VMEM((npad,), jnp.float32),
            pltpu.VMEM((npad,), jnp.int32),
            pltpu.VMEM((npad,), jnp.float32),
            pltpu.VMEM((npad,), jnp.int32),
            pltpu.VMEM((npad,), jnp.float32),
            pltpu.VMEM((npad,), jnp.float32),
            pltpu.VMEM((acc_rows, d), jnp.float32),
            pltpu.VMEM((d,), jnp.float32),
            pltpu.SMEM((1,), jnp.int32),
            pltpu.SemaphoreType.DMA,
        ],
    )(simp, rmax.reshape(b, npad), rarg.reshape(b, npad), slots)

    return merged, mt_pad[:, :n]


# speedup vs baseline: 4.6024x; 1.5942x over previous
import functools

import jax
import jax.numpy as jnp
from jax import lax
from jax.experimental import pallas as pl
from jax.experimental.pallas import tpu as pltpu
from jax.experimental.pallas import tpu_sc as plsc

_THRESHOLD = 0.9
_L = 16
_BIG = 0x7FFFFFFF


def _sim_body(slots_ref, simp_ref, rmax_ref, rarg_ref):
    n = slots_ref.shape[1]
    npad = simp_ref.shape[2]
    x = slots_ref[0]
    nrm = jnp.sqrt(jnp.sum(x * x, axis=1, keepdims=True))
    xn = x / jnp.maximum(nrm, 1e-12)
    sim = lax.dot_general(
        xn, xn, (((1,), (1,)), ((), ())),
        preferred_element_type=jnp.float32,
    )
    row_i = lax.broadcasted_iota(jnp.int32, (n, n), 0)
    col_i = lax.broadcasted_iota(jnp.int32, (n, n), 1)
    sim = jnp.where(row_i == col_i, sim - 2.0, sim)

    cmax = jnp.max(sim, axis=0, keepdims=True)
    pad = npad - n
    rmax_ref[0] = jnp.concatenate(
        [cmax, jnp.full((1, pad), -2.0, jnp.float32)], axis=1)

    @pl.when(jnp.max(cmax) > _THRESHOLD)
    def _():
        carg = jnp.min(jnp.where(sim == cmax, row_i, jnp.int32(_BIG)),
                       axis=0, keepdims=True)
        simp_ref[0] = jnp.concatenate(
            [sim, jnp.full((n, pad), -2.0, jnp.float32)], axis=1)
        rarg_ref[0] = jnp.concatenate(
            [carg, jnp.zeros((1, pad), jnp.int32)], axis=1)


def _greedy_body(simp, rmax0, rarg0, slots, mt_out, merged,
                 rmax_v, rarg_v, alive_v, mt_v, row_v, cnt_v, acc_v, srow_v,
                 mflag, sem):
    b_total, n_rows, npad = simp.shape
    d = slots.shape[2]
    nchunks = npad // _L
    dchunks = d // _L
    wid = lax.axis_index("s") * 2 + lax.axis_index("c")
    lane = lax.iota(jnp.int32, _L)
    lane0 = lane == 0

    def splat_i(s):
        return jnp.full((_L,), s, jnp.int32)

    def splat_f(s):
        return jnp.full((_L,), s, jnp.float32)

    def gather_i(ref, idx):
        return jnp.min(plsc.load_gather(ref, [splat_i(idx)]))

    @pl.when(wid < b_total)
    def _():
        b = wid
        pltpu.async_copy(rmax0.at[b], rmax_v, sem).wait()
        pltpu.async_copy(rarg0.at[b], rarg_v, sem).wait()

        def init_chunk(j, _):
            idx = lane + j * _L
            alive_v[pl.ds(j * _L, _L)] = jnp.where(
                idx < n_rows, 1.0, 0.0).astype(jnp.float32)
            mt_v[pl.ds(j * _L, _L)] = idx
            return 0

        lax.fori_loop(0, nchunks, init_chunk, 0)

        def global_argmax():
            def step(j, carry):
                vm, vi = carry
                v = rmax_v[pl.ds(j * _L, _L)]
                idx = lane + j * _L
                upd = v > vm
                return jnp.where(upd, v, vm), jnp.where(upd, idx, vi)

            vm, vi = lax.fori_loop(
                0, nchunks, step,
                (splat_f(-3.0), jnp.zeros((_L,), jnp.int32)))
            m = jnp.max(vm)
            r = jnp.min(jnp.where(vm == m, vi, jnp.int32(_BIG)))
            return m, r

        def masked_row_argmax():
            def step(k, carry):
                vm, vi = carry
                v = row_v[pl.ds(k * _L, _L)]
                a = alive_v[pl.ds(k * _L, _L)]
                w = jnp.where(a > 0.5, v, -2.0)
                idx = lane + k * _L
                upd = w > vm
                return jnp.where(upd, w, vm), jnp.where(upd, idx, vi)

            vm, vi = lax.fori_loop(
                0, nchunks, step,
                (splat_f(-3.0), jnp.zeros((_L,), jnp.int32)))
            m = jnp.max(vm)
            c = jnp.min(jnp.where(vm == m, vi, jnp.int32(_BIG)))
            return m, c

        def cond(st):
            return st != 0

        def body(st):
            m, r = global_argmax()
            go = m > _THRESHOLD
            _do_merge(go, r)
            return go.astype(jnp.int32)

        def _do_merge(go, r):
            @pl.when(go)
            def _():
                mflag[0] = jnp.int32(1)
                c = gather_i(rarg_v, r)
                src = jnp.maximum(r, c)
                tgt = jnp.minimum(r, c)
                plsc.store_scatter(mt_v, [splat_i(src)], splat_i(tgt),
                                   mask=lane0)
                plsc.store_scatter(alive_v, [splat_i(src)], splat_f(0.0),
                                   mask=lane0)
                plsc.store_scatter(rmax_v, [splat_i(src)], splat_f(-2.0),
                                   mask=lane0)

                def scan_chunk(j, _):
                    rarg_c = rarg_v[pl.ds(j * _L, _L)]
                    alive_c = alive_v[pl.ds(j * _L, _L)]
                    need = (rarg_c == src) & (alive_c > 0.5)
                    cnt = plsc.all_reduce_population_count(need)

                    @pl.when(jnp.max(cnt) > 0)
                    def _():
                        def lane_step(l, _):
                            i = j * _L + l
                            hit = (gather_i(rarg_v, i) == src) & (
                                jnp.min(plsc.load_gather(
                                    alive_v, [splat_i(i)])) > 0.5)

                            @pl.when(hit)
                            def _():
                                pltpu.async_copy(simp.at[b, i], row_v,
                                                 sem).wait()
                                nm, nc = masked_row_argmax()
                                plsc.store_scatter(
                                    rmax_v, [splat_i(i)], splat_f(nm),
                                    mask=lane0)
                                plsc.store_scatter(
                                    rarg_v, [splat_i(i)], splat_i(nc),
                                    mask=lane0)

                            return 0

                        lax.fori_loop(0, _L, lane_step, 0)

                    return 0

                lax.fori_loop(0, nchunks, scan_chunk, 0)

        mflag[0] = jnp.int32(0)
        lax.while_loop(cond, body, jnp.int32(1))
        pltpu.async_copy(mt_v, mt_out.at[b], sem).wait()

    sax = lax.axis_index("s")
    cax = lax.axis_index("c")

    @pl.when(sax >= 2)
    def _():
        cb = cax + 2 * ((sax - 2) // 7)
        sl_idx = (sax - 2) % 7

        @pl.when(sl_idx < 6)
        def _():
            src = slots.at[cb, pl.ds(sl_idx * 144, 144)]
            dst = merged.at[cb, pl.ds(sl_idx * 144, 144)]
            stage = acc_v.at[pl.ds(0, 144)]
            pltpu.async_copy(src, stage, sem).wait()
            pltpu.async_copy(stage, dst, sem).wait()

        @pl.when(sl_idx == 6)
        def _():
            src = slots.at[cb, pl.ds(864, 136)]
            dst = merged.at[cb, pl.ds(864, 136)]
            stage = acc_v.at[pl.ds(0, 136)]
            pltpu.async_copy(src, stage, sem).wait()
            pltpu.async_copy(stage, dst, sem).wait()

    plsc.subcore_barrier()

    @pl.when(wid < b_total)
    def _():
        b = wid

        @pl.when(mflag[0] != 0)
        def _():
            ones = jnp.ones((_L,), jnp.float32)
            zeros = jnp.zeros((_L,), jnp.float32)

            def zc(j, _):
                cnt_v[pl.ds(j * _L, _L)] = zeros
                return 0

            lax.fori_loop(0, nchunks, zc, 0)

            def cc(j, _):
                mtc = mt_v[pl.ds(j * _L, _L)]
                valid = (lane + j * _L) < n_rows
                plsc.addupdate_scatter(cnt_v, [mtc], ones, mask=valid)
                return 0

            lax.fori_loop(0, nchunks, cc, 0)

            rpp = acc_v.shape[0]
            for p in range((n_rows + rpp - 1) // rpp):
                base = p * rpp
                rows_out = min(rpp, n_rows - base)

                def za(r, _):
                    def zk(k, _):
                        acc_v[r, pl.ds(k * _L, _L)] = zeros
                        return 0

                    lax.fori_loop(0, dchunks, zk, 0)
                    return 0

                lax.fori_loop(0, rpp, za, 0)

                def si(i, _):
                    t = gather_i(mt_v, i)
                    hit = (t >= base) & (t < base + rpp)

                    @pl.when(hit)
                    def _():
                        pltpu.async_copy(slots.at[b, i], srow_v, sem).wait()
                        rel = t - base

                        def ak(k, _):
                            sl = pl.ds(k * _L, _L)
                            acc_v[rel, sl] = acc_v[rel, sl] + srow_v[sl]
                            return 0

                        lax.fori_loop(0, dchunks, ak, 0)

                    return 0

                lax.fori_loop(0, n_rows, si, 0)

                def dv(r, _):
                    cv = plsc.load_gather(cnt_v, [splat_i(base + r)])
                    scale = ones / jnp.maximum(cv, 1.0)

                    def mk(k, _):
                        sl = pl.ds(k * _L, _L)
                        acc_v[r, sl] = acc_v[r, sl] * scale
                        return 0

                    lax.fori_loop(0, dchunks, mk, 0)
                    return 0

                lax.fori_loop(0, rows_out, dv, 0)
                pltpu.async_copy(acc_v.at[pl.ds(0, rows_out)],
                                 merged.at[b, pl.ds(base, rows_out)],
                                 sem).wait()


def kernel(slots):
    b, n, d = slots.shape
    npad = ((n + _L - 1) // _L) * _L

    simp, rmax, rarg = pl.pallas_call(
        _sim_body,
        grid=(b,),
        in_specs=[pl.BlockSpec((1, n, d), lambda i: (i, 0, 0))],
        out_specs=[
            pl.BlockSpec((1, n, npad), lambda i: (i, 0, 0)),
            pl.BlockSpec((1, 1, npad), lambda i: (i, 0, 0)),
            pl.BlockSpec((1, 1, npad), lambda i: (i, 0, 0)),
        ],
        out_shape=[
            jax.ShapeDtypeStruct((b, n, npad), jnp.float32),
            jax.ShapeDtypeStruct((b, 1, npad), jnp.float32),
            jax.ShapeDtypeStruct((b, 1, npad), jnp.int32),
        ],
    )(slots)

    mesh = plsc.VectorSubcoreMesh(core_axis_name="c", subcore_axis_name="s",
                                  num_cores=2, num_subcores=16)
    acc_rows = 256
    mt_pad, merged = pl.kernel(
        _greedy_body,
        out_type=[
            jax.ShapeDtypeStruct((b, npad), jnp.int32),
            jax.ShapeDtypeStruct((b, n, d), jnp.float32),
        ],
        mesh=mesh,
        compiler_params=pltpu.CompilerParams(needs_layout_passes=False),
        scratch_types=[
            pltpu.VMEM((npad,), jnp.float32),
            pltpu.VMEM((npad,), jnp.int32),
            pltpu.VMEM((npad,), jnp.float32),
            pltpu.VMEM((npad,), jnp.int32),
            pltpu.VMEM((npad,), jnp.float32),
            pltpu.VMEM((npad,), jnp.float32),
            pltpu.VMEM((acc_rows, d), jnp.float32),
            pltpu.VMEM((d,), jnp.float32),
            pltpu.SMEM((1,), jnp.int32),
            pltpu.SemaphoreType.DMA,
        ],
    )(simp, rmax.reshape(b, npad), rarg.reshape(b, npad), slots)

    return merged, mt_pad[:, :n]
